# baseline (speedup 1.0000x reference)
import jax
import jax.numpy as jnp
from jax import lax
from jax.experimental import pallas as pl
from jax.experimental.pallas import tpu as pltpu
from jax.experimental.pallas import tpu_sc as plsc

EMB_DIM = 64
BATCH = 16384
NUM_CORES = 2
NUM_SUBCORES = 16
NW = NUM_CORES * NUM_SUBCORES
PAIRS_PER_W = BATCH // NW
CHUNK = 128
NCHUNK = PAIRS_PER_W // CHUNK

TC_CHUNK = 2048
TC_STEPS = BATCH // TC_CHUNK


def _sc_gather_body(idx_all, ce, we, out, idx_v, rows_v, sem):
    wid = lax.axis_index("s") * NUM_CORES + lax.axis_index("c")
    base = wid * PAIRS_PER_W
    crow = wid * NCHUNK

    tables = (ce, we, ce, we)
    for a, table in enumerate(tables):
        pltpu.sync_copy(idx_all.at[a, pl.ds(crow, NCHUNK)], idx_v)
        copies = []
        for j in range(NCHUNK):
            copies.append(
                pltpu.async_copy(table.at[idx_v.at[j]],
                                 rows_v.at[pl.ds(j * CHUNK, CHUNK)], sem))
        for cp in copies:
            cp.wait()
        pltpu.sync_copy(rows_v, out.at[a, pl.ds(base, PAIRS_PER_W)])


def _sc_gather(idx_all, center_emb, window_emb):
    mesh = plsc.VectorSubcoreMesh(core_axis_name="c", subcore_axis_name="s",
                                  num_cores=NUM_CORES,
                                  num_subcores=NUM_SUBCORES)
    return pl.kernel(
        _sc_gather_body,
        out_type=jax.ShapeDtypeStruct((4, BATCH, EMB_DIM), jnp.float32),
        mesh=mesh,
        compiler_params=pltpu.CompilerParams(use_tc_tiling_on_sc=False),
        scratch_types=[
            pltpu.VMEM((NCHUNK, CHUNK), jnp.int32),
            pltpu.VMEM((PAIRS_PER_W, EMB_DIM), jnp.float32),
            pltpu.SemaphoreType.DMA,
        ],
    )(idx_all, center_emb, window_emb)


def _loss_body(rows_ref, o_ref):
    i = pl.program_id(0)
    pc = rows_ref[0]
    pw = rows_ref[1]
    nc = rows_ref[2]
    nw = rows_ref[3]
    pos_score = jnp.sum(pc * pw, axis=1)
    neg_score = jnp.sum(nc * nw, axis=1)

    def logsig(x):
        return jnp.minimum(x, 0.0) - jnp.log1p(jnp.exp(-jnp.abs(x)))

    part = jnp.sum(logsig(pos_score)) + jnp.sum(logsig(-neg_score))

    @pl.when(i == 0)
    def _():
        o_ref[0, 0] = 0.0

    o_ref[0, 0] -= part


def _loss_kernel(rows):
    return pl.pallas_call(
        _loss_body,
        grid=(TC_STEPS,),
        in_specs=[pl.BlockSpec((4, TC_CHUNK, EMB_DIM), lambda i: (0, i, 0))],
        out_specs=pl.BlockSpec(memory_space=pltpu.SMEM),
        out_shape=jax.ShapeDtypeStruct((1, 1), jnp.float32),
    )(rows)


def kernel(pos_center, pos_window, neg_center, neg_window, center_emb,
           window_emb):
    idx_all = jnp.stack(
        [pos_center, pos_window, neg_center, neg_window]
    ).reshape(4, BATCH // CHUNK, CHUNK)
    rows = _sc_gather(idx_all, center_emb, window_emb)
    loss = _loss_kernel(rows)
    return loss[0, 0]


# speedup vs baseline: 2.5634x; 2.5634x over previous
import jax
import jax.numpy as jnp
from jax import lax
from jax.experimental import pallas as pl
from jax.experimental.pallas import tpu as pltpu
from jax.experimental.pallas import tpu_sc as plsc

EMB_DIM = 64
VOCAB = 1999999
BATCH = 16384
NUM_CORES = 2
NUM_SUBCORES = 16
NW = NUM_CORES * NUM_SUBCORES
PAIRS_PER_W = BATCH // NW
CHUNK = 128
NCHUNK = PAIRS_PER_W // CHUNK

TR_BN = 16384
TR_STEPS = -(-VOCAB // TR_BN)
VOCAB_PAD = TR_STEPS * TR_BN

TC_CHUNK = 2048
TC_STEPS = BATCH // TC_CHUNK


def _tr_body(ce_ref, we_ref, o_ref):
    eye = (lax.broadcasted_iota(jnp.int32, (EMB_DIM, EMB_DIM), 0)
           == lax.broadcasted_iota(jnp.int32, (EMB_DIM, EMB_DIM), 1)
           ).astype(jnp.float32)
    dn = (((0,), (0,)), ((), ()))
    o_ref[:, 0:EMB_DIM] = lax.dot_general(
        ce_ref[...], eye, dn, preferred_element_type=jnp.float32)
    o_ref[:, EMB_DIM:2 * EMB_DIM] = lax.dot_general(
        we_ref[...], eye, dn, preferred_element_type=jnp.float32)


def _transpose_combine(center_emb, window_emb):
    return pl.pallas_call(
        _tr_body,
        grid=(TR_STEPS,),
        in_specs=[
            pl.BlockSpec((EMB_DIM, TR_BN), lambda i: (0, i)),
            pl.BlockSpec((EMB_DIM, TR_BN), lambda i: (0, i)),
        ],
        out_specs=pl.BlockSpec((TR_BN, 2 * EMB_DIM), lambda i: (i, 0)),
        out_shape=jax.ShapeDtypeStruct((VOCAB_PAD, 2 * EMB_DIM), jnp.float32),
    )(center_emb.T, window_emb.T)


def _sc_gather_body(idx_all, combined, out, idx_v, rows_v, sem):
    wid = lax.axis_index("s") * NUM_CORES + lax.axis_index("c")
    pltpu.sync_copy(idx_all.at[wid], idx_v)
    for a in range(4):
        copies = []
        for j in range(NCHUNK):
            copies.append(
                pltpu.async_copy(combined.at[idx_v.at[a, j]],
                                 rows_v.at[pl.ds(j * CHUNK, CHUNK)], sem))
        for cp in copies:
            cp.wait()
        pltpu.sync_copy(rows_v, out.at[a, pl.ds(wid * PAIRS_PER_W,
                                                PAIRS_PER_W)])


def _sc_gather(idx_all, combined):
    mesh = plsc.VectorSubcoreMesh(core_axis_name="c", subcore_axis_name="s",
                                  num_cores=NUM_CORES,
                                  num_subcores=NUM_SUBCORES)
    return pl.kernel(
        _sc_gather_body,
        out_type=jax.ShapeDtypeStruct((4, BATCH, 2 * EMB_DIM), jnp.float32),
        mesh=mesh,
        compiler_params=pltpu.CompilerParams(use_tc_tiling_on_sc=True),
        scratch_types=[
            pltpu.VMEM((4, NCHUNK, CHUNK), jnp.int32),
            pltpu.VMEM((PAIRS_PER_W, 2 * EMB_DIM), jnp.float32),
            pltpu.SemaphoreType.DMA,
        ],
    )(idx_all, combined)


def _loss_body(rows_ref, o_ref):
    i = pl.program_id(0)
    pc = rows_ref[0, :, 0:EMB_DIM]
    pw = rows_ref[1, :, EMB_DIM:2 * EMB_DIM]
    nc = rows_ref[2, :, 0:EMB_DIM]
    nw = rows_ref[3, :, EMB_DIM:2 * EMB_DIM]
    pos_score = jnp.sum(pc * pw, axis=1)
    neg_score = jnp.sum(nc * nw, axis=1)

    def logsig(x):
        return jnp.minimum(x, 0.0) - jnp.log1p(jnp.exp(-jnp.abs(x)))

    part = jnp.sum(logsig(pos_score)) + jnp.sum(logsig(-neg_score))

    @pl.when(i == 0)
    def _():
        o_ref[0, 0] = 0.0

    o_ref[0, 0] -= part


def _loss_kernel(rows):
    return pl.pallas_call(
        _loss_body,
        grid=(TC_STEPS,),
        in_specs=[pl.BlockSpec((4, TC_CHUNK, 2 * EMB_DIM),
                               lambda i: (0, i, 0))],
        out_specs=pl.BlockSpec(memory_space=pltpu.SMEM),
        out_shape=jax.ShapeDtypeStruct((1, 1), jnp.float32),
    )(rows)


def kernel(pos_center, pos_window, neg_center, neg_window, center_emb,
           window_emb):
    combined = _transpose_combine(center_emb, window_emb)
    idx_all = (jnp.stack([pos_center, pos_window, neg_center, neg_window])
               .reshape(4, NW, NCHUNK, CHUNK)
               .transpose(1, 0, 2, 3))
    rows = _sc_gather(idx_all, combined)
    loss = _loss_kernel(rows)
    return loss[0, 0]
